# baseline (speedup 1.0000x reference)
import functools

import jax
import jax.numpy as jnp
from jax import lax
from jax.experimental import pallas as pl
from jax.experimental.pallas import tpu as pltpu
from jax.experimental.pallas import tpu_sc as plsc

_N = 10000
_D = 128
_E = 320000
_NC = 2
_NS = 16
_NW = _NC * _NS
_BE = 32
_NCH = 320
_HCH = 40
_NB = 8
_LOOK = 4
_EPW = _NCH * _BE
_EP = _NW * _EPW
_ZR = 640
_NPAD = _NS * _ZR


def _segment_sum_partials(h, srcr, dstr, zeros):
    mesh = plsc.VectorSubcoreMesh(core_axis_name="c", subcore_axis_name="s")

    @functools.partial(
        pl.kernel,
        out_type=jax.ShapeDtypeStruct((_NC, _NPAD, _D), jnp.float32),
        mesh=mesh,
        scratch_types=[
            pltpu.VMEM((_HCH, _BE), jnp.int32),
            pltpu.VMEM((_HCH, _BE), jnp.int32),
            *([pltpu.VMEM((_BE, _D), jnp.float32)] * _NB),
            pltpu.VMEM_SHARED((_NPAD, _D), jnp.float32),
            pltpu.SemaphoreType.DMA,
            *([pltpu.SemaphoreType.DMA] * _NB),
            *([pltpu.SemaphoreType.DMA] * _NB),
        ],
    )
    def k(h_hbm, src_hbm, dst_hbm, zero_hbm, out_hbm, src_v, dst_v, *rest):
        bufs = rest[:_NB]
        agg_s = rest[_NB]
        sem0 = rest[_NB + 1]
        gsem = rest[_NB + 2:2 * _NB + 2]
        asem = rest[2 * _NB + 2:]
        c = lax.axis_index("c")
        s = lax.axis_index("s")
        wid = s * _NC + c
        gs0, as0 = gsem[0], asem[0]

        def stage_idx(half):
            ssrc = src_hbm.at[wid, pl.ds(half * _HCH, _HCH)]
            sdst = dst_hbm.at[wid, pl.ds(half * _HCH, _HCH)]
            pltpu.async_copy(ssrc, src_v, gs0)
            pltpu.async_copy(sdst, dst_v, as0)
            pltpu.make_async_copy(ssrc, src_v, gs0).wait()
            pltpu.make_async_copy(sdst, dst_v, as0).wait()

        zsrc = zero_hbm.at[pl.ds(s * _ZR, _ZR)]
        zdst = agg_s.at[pl.ds(s * _ZR, _ZR)]
        pltpu.async_copy(zsrc, zdst, sem0)
        stage_idx(0)
        pltpu.make_async_copy(zsrc, zdst, sem0).wait()
        plsc.subcore_barrier()

        def fire_gather(jj, b):
            pltpu.async_copy(h_hbm.at[src_v.at[jj]], bufs[b], gsem[b])

        def gather_done(jj, b):
            pltpu.make_async_copy(h_hbm.at[src_v.at[jj]], bufs[b], gsem[b]).wait()

        def fire_add(jj, b):
            pltpu.async_copy(bufs[b], agg_s.at[dst_v.at[jj]], asem[b], add=True)

        def add_done(jj, b):
            pltpu.make_async_copy(bufs[b], agg_s.at[dst_v.at[jj]], asem[b]).wait()

        def run_slab():
            for u in range(_LOOK):
                fire_gather(u, u)

            @pl.loop(0, _HCH, step=_NB)
            def _(j):
                for u in range(_NB):
                    jj = j + u
                    gather_done(jj, u)
                    fire_add(jj, u)
                    bl = (u + _LOOK) % _NB
                    if u < _LOOK:
                        @pl.when(j >= _LOOK - u)
                        def _():
                            add_done(jj + _LOOK - _NB, bl)

                        fire_gather(jj + _LOOK, bl)
                    else:
                        add_done(jj + _LOOK - _NB, bl)

                        @pl.when(jj + _LOOK < _HCH)
                        def _():
                            fire_gather(jj + _LOOK, bl)

            for t in range(_HCH - _LOOK, _HCH):
                add_done(t, t % _NB)

        run_slab()
        for q in range(1, _NCH // _HCH):
            stage_idx(q)
            run_slab()
        plsc.subcore_barrier()
        pltpu.sync_copy(agg_s.at[pl.ds(s * _ZR, _ZR)],
                        out_hbm.at[c, pl.ds(s * _ZR, _ZR)])

    return k(h, srcr, dstr, zeros)


def _mlp(h, p0, p1, Wa, ba, Wb, bb, relu_out):
    R = 1000

    def body(h_ref, p0_ref, p1_ref, wa_ref, ba_ref, wb_ref, bb_ref, o_ref):
        m = h_ref[...] + p0_ref[...] + p1_ref[...]
        hid = jnp.dot(m, wa_ref[...], preferred_element_type=jnp.float32) + ba_ref[...]
        hid = jnp.maximum(hid, 0.0)
        o = jnp.dot(hid, wb_ref[...], preferred_element_type=jnp.float32) + bb_ref[...]
        if relu_out:
            o = jnp.maximum(o, 0.0)
        o_ref[...] = o

    bs_rows = pl.BlockSpec((R, _D), lambda i: (i, 0))
    bs_w = pl.BlockSpec((_D, _D), lambda i: (0, 0))
    bs_b = pl.BlockSpec((1, _D), lambda i: (0, 0))
    return pl.pallas_call(
        body,
        grid=(_N // R,),
        in_specs=[bs_rows, bs_rows, bs_rows, bs_w, bs_b, bs_w, bs_b],
        out_specs=bs_rows,
        out_shape=jax.ShapeDtypeStruct((_N, _D), jnp.float32),
    )(h, p0, p1, Wa, ba.reshape(1, _D), Wb, bb.reshape(1, _D))


def kernel(x, edge_index, W0a, b0a, W0b, b0b, W1a, b1a, W1b, b1b, W2a, b2a, W2b, b2b):
    src = edge_index[0]
    dst = edge_index[1]
    pad = _EP - _E
    pad_i = jnp.arange(pad, dtype=jnp.int32)
    srcr = jnp.concatenate([src, pad_i % _N]).reshape(_NW, _NCH, _BE)
    dstr = jnp.concatenate([dst, _N + pad_i % (_NPAD - _N)]).reshape(_NW, _NCH, _BE)
    zeros = jnp.zeros((_NPAD, _D), jnp.float32)

    h = x
    for Wa, ba, Wb, bb, relu_out in (
        (W0a, b0a, W0b, b0b, True),
        (W1a, b1a, W1b, b1b, True),
        (W2a, b2a, W2b, b2b, False),
    ):
        parts = _segment_sum_partials(h, srcr, dstr, zeros)
        h = _mlp(h, parts[0, :_N], parts[1, :_N], Wa, ba, Wb, bb, relu_out)
    return h


# speedup vs baseline: 1.0797x; 1.0797x over previous
import functools

import jax
import jax.numpy as jnp
from jax import lax
from jax.experimental import pallas as pl
from jax.experimental.pallas import tpu as pltpu
from jax.experimental.pallas import tpu_sc as plsc

_N = 10000
_D = 128
_E = 320000
_NC = 2
_NS = 16
_NW = _NC * _NS
_BE = 64
_NCH = 160
_HCH = 40
_NB = 4
_LOOK = 2
_EPW = _NCH * _BE
_EP = _NW * _EPW
_ZR = 640
_NPAD = _NS * _ZR


def _segment_sum_partials(h, srcr, dstr, zeros):
    mesh = plsc.VectorSubcoreMesh(core_axis_name="c", subcore_axis_name="s")

    @functools.partial(
        pl.kernel,
        out_type=jax.ShapeDtypeStruct((_NC, _NPAD, _D), jnp.float32),
        mesh=mesh,
        scratch_types=[
            pltpu.VMEM((_HCH, _BE), jnp.int32),
            pltpu.VMEM((_HCH, _BE), jnp.int32),
            *([pltpu.VMEM((_BE, _D), jnp.float32)] * _NB),
            pltpu.VMEM_SHARED((_NPAD, _D), jnp.float32),
            pltpu.SemaphoreType.DMA,
            *([pltpu.SemaphoreType.DMA] * _NB),
            *([pltpu.SemaphoreType.DMA] * _NB),
        ],
    )
    def k(h_hbm, src_hbm, dst_hbm, zero_hbm, out_hbm, src_v, dst_v, *rest):
        bufs = rest[:_NB]
        agg_s = rest[_NB]
        sem0 = rest[_NB + 1]
        gsem = rest[_NB + 2:2 * _NB + 2]
        asem = rest[2 * _NB + 2:]
        c = lax.axis_index("c")
        s = lax.axis_index("s")
        wid = s * _NC + c
        gs0, as0 = gsem[0], asem[0]

        def stage_idx(half):
            ssrc = src_hbm.at[wid, pl.ds(half * _HCH, _HCH)]
            sdst = dst_hbm.at[wid, pl.ds(half * _HCH, _HCH)]
            pltpu.async_copy(ssrc, src_v, gs0)
            pltpu.async_copy(sdst, dst_v, as0)
            pltpu.make_async_copy(ssrc, src_v, gs0).wait()
            pltpu.make_async_copy(sdst, dst_v, as0).wait()

        zsrc = zero_hbm.at[pl.ds(s * _ZR, _ZR)]
        zdst = agg_s.at[pl.ds(s * _ZR, _ZR)]
        pltpu.async_copy(zsrc, zdst, sem0)
        stage_idx(0)
        pltpu.make_async_copy(zsrc, zdst, sem0).wait()
        plsc.subcore_barrier()

        def fire_gather(jj, b):
            pltpu.async_copy(h_hbm.at[src_v.at[jj]], bufs[b], gsem[b])

        def gather_done(jj, b):
            pltpu.make_async_copy(h_hbm.at[src_v.at[jj]], bufs[b], gsem[b]).wait()

        def fire_add(jj, b):
            pltpu.async_copy(bufs[b], agg_s.at[dst_v.at[jj]], asem[b], add=True)

        def add_done(jj, b):
            pltpu.make_async_copy(bufs[b], agg_s.at[dst_v.at[jj]], asem[b]).wait()

        def run_slab():
            fire_gather(0, 0)
            fire_gather(1, 1)
            fire_gather(2, 2)
            fire_gather(3, 3)

            @pl.loop(0, _HCH, step=4)
            def _(j):
                for b in range(2):
                    gather_done(j + b, b)
                    fire_add(j + b, b)
                for b in range(2):
                    add_done(j + b, b)

                @pl.when(j + 4 < _HCH)
                def _():
                    fire_gather(j + 4, 0)
                    fire_gather(j + 5, 1)

                for b in range(2):
                    gather_done(j + 2 + b, 2 + b)
                    fire_add(j + 2 + b, 2 + b)
                for b in range(2):
                    add_done(j + 2 + b, 2 + b)

                @pl.when(j + 6 < _HCH)
                def _():
                    fire_gather(j + 6, 2)
                    fire_gather(j + 7, 3)

        run_slab()
        for q in range(1, _NCH // _HCH):
            stage_idx(q)
            run_slab()
        plsc.subcore_barrier()
        pltpu.sync_copy(agg_s.at[pl.ds(s * _ZR, _ZR)],
                        out_hbm.at[c, pl.ds(s * _ZR, _ZR)])

    return k(h, srcr, dstr, zeros)


def _mlp(h, p0, p1, Wa, ba, Wb, bb, relu_out):
    R = 1000

    def body(h_ref, p0_ref, p1_ref, wa_ref, ba_ref, wb_ref, bb_ref, o_ref):
        m = h_ref[...] + p0_ref[...] + p1_ref[...]
        hid = jnp.dot(m, wa_ref[...], preferred_element_type=jnp.float32) + ba_ref[...]
        hid = jnp.maximum(hid, 0.0)
        o = jnp.dot(hid, wb_ref[...], preferred_element_type=jnp.float32) + bb_ref[...]
        if relu_out:
            o = jnp.maximum(o, 0.0)
        o_ref[...] = o

    bs_rows = pl.BlockSpec((R, _D), lambda i: (i, 0))
    bs_w = pl.BlockSpec((_D, _D), lambda i: (0, 0))
    bs_b = pl.BlockSpec((1, _D), lambda i: (0, 0))
    return pl.pallas_call(
        body,
        grid=(_N // R,),
        in_specs=[bs_rows, bs_rows, bs_rows, bs_w, bs_b, bs_w, bs_b],
        out_specs=bs_rows,
        out_shape=jax.ShapeDtypeStruct((_N, _D), jnp.float32),
    )(h, p0, p1, Wa, ba.reshape(1, _D), Wb, bb.reshape(1, _D))


def kernel(x, edge_index, W0a, b0a, W0b, b0b, W1a, b1a, W1b, b1b, W2a, b2a, W2b, b2b):
    src = edge_index[0]
    dst = edge_index[1]
    pad = _EP - _E
    pad_i = jnp.arange(pad, dtype=jnp.int32)
    srcr = jnp.concatenate([src, pad_i % _N]).reshape(_NW, _NCH, _BE)
    dstr = jnp.concatenate([dst, _N + pad_i % (_NPAD - _N)]).reshape(_NW, _NCH, _BE)
    zeros = jnp.zeros((_NPAD, _D), jnp.float32)

    h = x
    for Wa, ba, Wb, bb, relu_out in (
        (W0a, b0a, W0b, b0b, True),
        (W1a, b1a, W1b, b1b, True),
        (W2a, b2a, W2b, b2b, False),
    ):
        parts = _segment_sum_partials(h, srcr, dstr, zeros)
        h = _mlp(h, parts[0, :_N], parts[1, :_N], Wa, ba, Wb, bb, relu_out)
    return h


# speedup vs baseline: 1.1333x; 1.0497x over previous
import functools

import jax
import jax.numpy as jnp
from jax import lax
from jax.experimental import pallas as pl
from jax.experimental.pallas import tpu as pltpu
from jax.experimental.pallas import tpu_sc as plsc

_N = 10000
_D = 128
_E = 320000
_NC = 2
_NS = 16
_NW = _NC * _NS
_BE = 64
_NCH = 160
_HCH = 40
_NB = 4
_LOOK = 2
_EPW = _NCH * _BE
_EP = _NW * _EPW
_ZR = 640
_NPAD = _NS * _ZR


def _segment_sum_partials(h, srcr, dstr, zeros):
    mesh = plsc.VectorSubcoreMesh(core_axis_name="c", subcore_axis_name="s")

    @functools.partial(
        pl.kernel,
        out_type=jax.ShapeDtypeStruct((_NC, _NPAD, _D), jnp.float32),
        mesh=mesh,
        scratch_types=[
            pltpu.VMEM((_HCH, _BE), jnp.int32),
            pltpu.VMEM((_HCH, _BE), jnp.int32),
            *([pltpu.VMEM((_BE, _D), jnp.float32)] * _NB),
            pltpu.VMEM_SHARED((_NPAD, _D), jnp.float32),
            pltpu.SemaphoreType.DMA,
            *([pltpu.SemaphoreType.DMA] * _NB),
            *([pltpu.SemaphoreType.DMA] * _NB),
        ],
    )
    def k(h_hbm, src_hbm, dst_hbm, zero_hbm, out_hbm, src_v, dst_v, *rest):
        bufs = rest[:_NB]
        agg_s = rest[_NB]
        sem0 = rest[_NB + 1]
        gsem = rest[_NB + 2:2 * _NB + 2]
        asem = rest[2 * _NB + 2:]
        c = lax.axis_index("c")
        s = lax.axis_index("s")
        wid = s * _NC + c
        gs0, as0 = gsem[0], asem[0]

        def stage_idx(half):
            ssrc = src_hbm.at[wid, pl.ds(half * _HCH, _HCH)]
            sdst = dst_hbm.at[wid, pl.ds(half * _HCH, _HCH)]
            pltpu.async_copy(ssrc, src_v, gs0)
            pltpu.async_copy(sdst, dst_v, as0)
            pltpu.make_async_copy(ssrc, src_v, gs0).wait()
            pltpu.make_async_copy(sdst, dst_v, as0).wait()

        zsrc = zero_hbm.at[pl.ds(s * _ZR, _ZR)]
        zdst = agg_s.at[pl.ds(s * _ZR, _ZR)]
        pltpu.async_copy(zsrc, zdst, sem0)
        stage_idx(0)
        pltpu.make_async_copy(zsrc, zdst, sem0).wait()
        plsc.subcore_barrier()

        def fire_gather(jj, b):
            pltpu.async_copy(h_hbm.at[src_v.at[jj]], bufs[b], gsem[b])

        def gather_done(jj, b):
            pltpu.make_async_copy(h_hbm.at[src_v.at[jj]], bufs[b], gsem[b]).wait()

        def fire_add(jj, b):
            pltpu.async_copy(bufs[b], agg_s.at[dst_v.at[jj]], asem[b], add=True)

        def add_done(jj, b):
            pltpu.make_async_copy(bufs[b], agg_s.at[dst_v.at[jj]], asem[b]).wait()

        def run_slab():
            fire_gather(0, 0)
            fire_gather(1, 1)
            fire_gather(2, 2)
            fire_gather(3, 3)

            @pl.loop(0, _HCH, step=4)
            def _(j):
                for b in range(2):
                    gather_done(j + b, b)
                    fire_add(j + b, b)
                for b in range(2):
                    add_done(j + b, b)

                @pl.when(j + 4 < _HCH)
                def _():
                    fire_gather(j + 4, 0)
                    fire_gather(j + 5, 1)

                for b in range(2):
                    gather_done(j + 2 + b, 2 + b)
                    fire_add(j + 2 + b, 2 + b)
                for b in range(2):
                    add_done(j + 2 + b, 2 + b)

                @pl.when(j + 6 < _HCH)
                def _():
                    fire_gather(j + 6, 2)
                    fire_gather(j + 7, 3)

        run_slab()
        for q in range(1, _NCH // _HCH):
            stage_idx(q)
            run_slab()
        plsc.subcore_barrier()
        pltpu.sync_copy(agg_s.at[pl.ds(s * _ZR, _ZR)],
                        out_hbm.at[c, pl.ds(s * _ZR, _ZR)])

    return k(h, srcr, dstr, zeros)


def _mlp(h, parts, Wa, ba, Wb, bb, relu_out):
    R = 1000

    def body(h_ref, p0_ref, p1_ref, wa_ref, ba_ref, wb_ref, bb_ref, o_ref):
        m = h_ref[...] + p0_ref[0] + p1_ref[0]
        hid = jnp.dot(m, wa_ref[...], preferred_element_type=jnp.float32) + ba_ref[...]
        hid = jnp.maximum(hid, 0.0)
        o = jnp.dot(hid, wb_ref[...], preferred_element_type=jnp.float32) + bb_ref[...]
        if relu_out:
            o = jnp.maximum(o, 0.0)
        o_ref[...] = o

    bs_rows = pl.BlockSpec((R, _D), lambda i: (i, 0))
    bs_p0 = pl.BlockSpec((1, R, _D), lambda i: (0, i, 0))
    bs_p1 = pl.BlockSpec((1, R, _D), lambda i: (1, i, 0))
    bs_w = pl.BlockSpec((_D, _D), lambda i: (0, 0))
    bs_b = pl.BlockSpec((1, _D), lambda i: (0, 0))
    return pl.pallas_call(
        body,
        grid=(_N // R,),
        in_specs=[bs_rows, bs_p0, bs_p1, bs_w, bs_b, bs_w, bs_b],
        out_specs=bs_rows,
        out_shape=jax.ShapeDtypeStruct((_N, _D), jnp.float32),
    )(h, parts, parts, Wa, ba.reshape(1, _D), Wb, bb.reshape(1, _D))


def kernel(x, edge_index, W0a, b0a, W0b, b0b, W1a, b1a, W1b, b1b, W2a, b2a, W2b, b2b):
    src = edge_index[0]
    dst = edge_index[1]
    pad = _EP - _E
    pad_i = jnp.arange(pad, dtype=jnp.int32)
    srcr = jnp.concatenate([src, pad_i % _N]).reshape(_NW, _NCH, _BE)
    dstr = jnp.concatenate([dst, _N + pad_i % (_NPAD - _N)]).reshape(_NW, _NCH, _BE)
    zeros = jnp.zeros((_NPAD, _D), jnp.float32)

    h = x
    for Wa, ba, Wb, bb, relu_out in (
        (W0a, b0a, W0b, b0b, True),
        (W1a, b1a, W1b, b1b, True),
        (W2a, b2a, W2b, b2b, False),
    ):
        parts = _segment_sum_partials(h, srcr, dstr, zeros)
        h = _mlp(h, parts, Wa, ba, Wb, bb, relu_out)
    return h


# speedup vs baseline: 1.1749x; 1.0366x over previous
import functools

import jax
import jax.numpy as jnp
from jax import lax
from jax.experimental import pallas as pl
from jax.experimental.pallas import tpu as pltpu
from jax.experimental.pallas import tpu_sc as plsc

_N = 10000
_D = 128
_E = 320000
_NC = 2
_NS = 16
_NW = _NC * _NS
_BE = 64
_NCH = 160
_HCH = 40
_NB = 4
_LOOK = 2
_EPW = _NCH * _BE
_EP = _NW * _EPW
_ZR = 640
_NPAD = _NS * _ZR


def _segment_sum_partials(h, srcr, dstr, zeros):
    mesh = plsc.VectorSubcoreMesh(core_axis_name="c", subcore_axis_name="s")

    @functools.partial(
        pl.kernel,
        out_type=jax.ShapeDtypeStruct((_NC, _NPAD, _D), jnp.float32),
        mesh=mesh,
        scratch_types=[
            pltpu.VMEM((_HCH, _BE), jnp.int32),
            pltpu.VMEM((_HCH, _BE), jnp.int32),
            *([pltpu.VMEM((_BE, _D), jnp.float32)] * _NB),
            pltpu.VMEM_SHARED((_NPAD, _D), jnp.float32),
            pltpu.SemaphoreType.DMA,
            *([pltpu.SemaphoreType.DMA] * _NB),
            *([pltpu.SemaphoreType.DMA] * _NB),
        ],
    )
    def k(h_hbm, src_hbm, dst_hbm, zero_hbm, out_hbm, src_v, dst_v, *rest):
        bufs = rest[:_NB]
        agg_s = rest[_NB]
        sem0 = rest[_NB + 1]
        gsem = rest[_NB + 2:2 * _NB + 2]
        asem = rest[2 * _NB + 2:]
        c = lax.axis_index("c")
        s = lax.axis_index("s")
        wid = s * _NC + c
        gs0, as0 = gsem[0], asem[0]

        def stage_idx(half):
            ssrc = src_hbm.at[wid, pl.ds(half * _HCH, _HCH)]
            sdst = dst_hbm.at[wid, pl.ds(half * _HCH, _HCH)]
            pltpu.async_copy(ssrc, src_v, gs0)
            pltpu.async_copy(sdst, dst_v, as0)
            pltpu.make_async_copy(ssrc, src_v, gs0).wait()
            pltpu.make_async_copy(sdst, dst_v, as0).wait()

        zsrc = zero_hbm.at[pl.ds(s * _ZR, _ZR)]
        zdst = agg_s.at[pl.ds(s * _ZR, _ZR)]
        pltpu.async_copy(zsrc, zdst, sem0)
        stage_idx(0)

        def fire_gather(jj, b):
            pltpu.async_copy(h_hbm.at[src_v.at[jj]], bufs[b], gsem[b])

        def gather_done(jj, b):
            pltpu.make_async_copy(h_hbm.at[src_v.at[jj]], bufs[b], gsem[b]).wait()

        def fire_add(jj, b):
            pltpu.async_copy(bufs[b], agg_s.at[dst_v.at[jj]], asem[b], add=True)

        def add_done(jj, b):
            pltpu.make_async_copy(bufs[b], agg_s.at[dst_v.at[jj]], asem[b]).wait()

        def prime():
            fire_gather(0, 0)
            fire_gather(1, 1)
            fire_gather(2, 2)
            fire_gather(3, 3)

        def run_slab():

            @pl.loop(0, _HCH, step=4)
            def _(j):
                for b in range(2):
                    gather_done(j + b, b)
                    fire_add(j + b, b)
                for b in range(2):
                    add_done(j + b, b)

                @pl.when(j + 4 < _HCH)
                def _():
                    fire_gather(j + 4, 0)
                    fire_gather(j + 5, 1)

                for b in range(2):
                    gather_done(j + 2 + b, 2 + b)
                    fire_add(j + 2 + b, 2 + b)
                for b in range(2):
                    add_done(j + 2 + b, 2 + b)

                @pl.when(j + 6 < _HCH)
                def _():
                    fire_gather(j + 6, 2)
                    fire_gather(j + 7, 3)

        prime()
        pltpu.make_async_copy(zsrc, zdst, sem0).wait()
        plsc.subcore_barrier()
        run_slab()
        for q in range(1, _NCH // _HCH):
            stage_idx(q)
            prime()
            run_slab()
        plsc.subcore_barrier()
        pltpu.sync_copy(agg_s.at[pl.ds(s * _ZR, _ZR)],
                        out_hbm.at[c, pl.ds(s * _ZR, _ZR)])

    return k(h, srcr, dstr, zeros)


def _mlp(h, parts, Wa, ba, Wb, bb, relu_out):
    R = 2000

    def body(h_ref, p0_ref, p1_ref, wa_ref, ba_ref, wb_ref, bb_ref, o_ref):
        m = h_ref[...] + p0_ref[0] + p1_ref[0]
        hid = jnp.dot(m, wa_ref[...], preferred_element_type=jnp.float32) + ba_ref[...]
        hid = jnp.maximum(hid, 0.0)
        o = jnp.dot(hid, wb_ref[...], preferred_element_type=jnp.float32) + bb_ref[...]
        if relu_out:
            o = jnp.maximum(o, 0.0)
        o_ref[...] = o

    bs_rows = pl.BlockSpec((R, _D), lambda i: (i, 0))
    bs_p0 = pl.BlockSpec((1, R, _D), lambda i: (0, i, 0))
    bs_p1 = pl.BlockSpec((1, R, _D), lambda i: (1, i, 0))
    bs_w = pl.BlockSpec((_D, _D), lambda i: (0, 0))
    bs_b = pl.BlockSpec((1, _D), lambda i: (0, 0))
    return pl.pallas_call(
        body,
        grid=(_N // R,),
        in_specs=[bs_rows, bs_p0, bs_p1, bs_w, bs_b, bs_w, bs_b],
        out_specs=bs_rows,
        out_shape=jax.ShapeDtypeStruct((_N, _D), jnp.float32),
    )(h, parts, parts, Wa, ba.reshape(1, _D), Wb, bb.reshape(1, _D))


def kernel(x, edge_index, W0a, b0a, W0b, b0b, W1a, b1a, W1b, b1b, W2a, b2a, W2b, b2b):
    src = edge_index[0]
    dst = edge_index[1]
    pad = _EP - _E
    pad_i = jnp.arange(pad, dtype=jnp.int32)
    srcr = jnp.concatenate([src, pad_i % _N]).reshape(_NW, _NCH, _BE)
    dstr = jnp.concatenate([dst, _N + pad_i % (_NPAD - _N)]).reshape(_NW, _NCH, _BE)
    zeros = jnp.zeros((_NPAD, _D), jnp.float32)

    h = x
    for Wa, ba, Wb, bb, relu_out in (
        (W0a, b0a, W0b, b0b, True),
        (W1a, b1a, W1b, b1b, True),
        (W2a, b2a, W2b, b2b, False),
    ):
        parts = _segment_sum_partials(h, srcr, dstr, zeros)
        h = _mlp(h, parts, Wa, ba, Wb, bb, relu_out)
    return h


# speedup vs baseline: 1.1995x; 1.0209x over previous
import functools

import jax
import jax.numpy as jnp
from jax import lax
from jax.experimental import pallas as pl
from jax.experimental.pallas import tpu as pltpu
from jax.experimental.pallas import tpu_sc as plsc

_N = 10000
_D = 128
_E = 320000
_NC = 2
_NS = 16
_NW = _NC * _NS
_BE = 64
_NCH = 160
_HCH = 16
_NB = 4
_LOOK = 2
_EPW = _NCH * _BE
_EP = _NW * _EPW
_ZR = 640
_NPAD = _NS * _ZR


def _segment_sum_partials(h, srcr, dstr, zeros):
    mesh = plsc.VectorSubcoreMesh(core_axis_name="c", subcore_axis_name="s")

    @functools.partial(
        pl.kernel,
        out_type=jax.ShapeDtypeStruct((_NC, _NPAD, _D), jnp.float32),
        mesh=mesh,
        scratch_types=[
            pltpu.VMEM((_HCH, _BE), jnp.int32),
            pltpu.VMEM((_HCH, _BE), jnp.int32),
            pltpu.VMEM((_HCH, _BE), jnp.int32),
            pltpu.VMEM((_HCH, _BE), jnp.int32),
            *([pltpu.VMEM((_BE, _D), jnp.float32)] * _NB),
            pltpu.VMEM_SHARED((_NPAD, _D), jnp.float32),
            pltpu.SemaphoreType.DMA,
            pltpu.SemaphoreType.DMA,
            pltpu.SemaphoreType.DMA,
            *([pltpu.SemaphoreType.DMA] * _NB),
            *([pltpu.SemaphoreType.DMA] * _NB),
        ],
    )
    def k(h_hbm, src_hbm, dst_hbm, zero_hbm, out_hbm,
          src_v0, dst_v0, src_v1, dst_v1, *rest):
        bufs = rest[:_NB]
        agg_s = rest[_NB]
        sem0, psem_s, psem_d = rest[_NB + 1:_NB + 4]
        gsem = rest[_NB + 4:2 * _NB + 4]
        asem = rest[2 * _NB + 4:]
        c = lax.axis_index("c")
        s = lax.axis_index("s")
        wid = s * _NC + c
        srcs = (src_v0, src_v1)
        dsts = (dst_v0, dst_v1)

        def stage_idx(q, wait):
            p = q % 2
            ssrc = src_hbm.at[wid, pl.ds(q * _HCH, _HCH)]
            sdst = dst_hbm.at[wid, pl.ds(q * _HCH, _HCH)]
            pltpu.async_copy(ssrc, srcs[p], psem_s)
            pltpu.async_copy(sdst, dsts[p], psem_d)
            if wait:
                stage_wait(q)

        def stage_wait(q):
            p = q % 2
            ssrc = src_hbm.at[wid, pl.ds(q * _HCH, _HCH)]
            sdst = dst_hbm.at[wid, pl.ds(q * _HCH, _HCH)]
            pltpu.make_async_copy(ssrc, srcs[p], psem_s).wait()
            pltpu.make_async_copy(sdst, dsts[p], psem_d).wait()

        zsrc = zero_hbm.at[pl.ds(s * _ZR, _ZR)]
        zdst = agg_s.at[pl.ds(s * _ZR, _ZR)]
        pltpu.async_copy(zsrc, zdst, sem0)
        stage_idx(0, wait=True)
        stage_idx(1, wait=False)

        def fire_gather(jj, b, p):
            pltpu.async_copy(h_hbm.at[srcs[p].at[jj]], bufs[b], gsem[b])

        def gather_done(jj, b, p):
            pltpu.make_async_copy(h_hbm.at[srcs[p].at[jj]], bufs[b], gsem[b]).wait()

        def fire_add(jj, b, p):
            pltpu.async_copy(bufs[b], agg_s.at[dsts[p].at[jj]], asem[b], add=True)

        def add_done(jj, b, p):
            pltpu.make_async_copy(bufs[b], agg_s.at[dsts[p].at[jj]], asem[b]).wait()

        def run_slab(q):
            p = q % 2
            last = q == _NCH // _HCH - 1

            @pl.loop(0, _HCH, step=4)
            def _(j):
                for b in range(2):
                    gather_done(j + b, b, p)
                    fire_add(j + b, b, p)
                for b in range(2):
                    add_done(j + b, b, p)

                @pl.when(j + 4 < _HCH)
                def _():
                    fire_gather(j + 4, 0, p)
                    fire_gather(j + 5, 1, p)

                if not last:
                    @pl.when(j + 4 >= _HCH)
                    def _():
                        stage_wait(q + 1)
                        fire_gather(j + 4 - _HCH, 0, 1 - p)
                        fire_gather(j + 5 - _HCH, 1, 1 - p)

                for b in range(2):
                    gather_done(j + 2 + b, 2 + b, p)
                    fire_add(j + 2 + b, 2 + b, p)
                for b in range(2):
                    add_done(j + 2 + b, 2 + b, p)

                @pl.when(j + 6 < _HCH)
                def _():
                    fire_gather(j + 6, 2, p)
                    fire_gather(j + 7, 3, p)

                if not last:
                    @pl.when(j + 6 >= _HCH)
                    def _():
                        fire_gather(j + 6 - _HCH, 2, 1 - p)
                        fire_gather(j + 7 - _HCH, 3, 1 - p)

        for b in range(_NB):
            fire_gather(b, b, 0)
        pltpu.make_async_copy(zsrc, zdst, sem0).wait()
        plsc.subcore_barrier()
        run_slab(0)
        for q in range(1, _NCH // _HCH):
            stage_idx(q + 1, wait=False) if q + 1 < _NCH // _HCH else None
            run_slab(q)
        plsc.subcore_barrier()
        pltpu.sync_copy(agg_s.at[pl.ds(s * _ZR, _ZR)],
                        out_hbm.at[c, pl.ds(s * _ZR, _ZR)])

    return k(h, srcr, dstr, zeros)


def _mlp(h, parts, Wa, ba, Wb, bb, relu_out):
    R = 2000

    def body(h_ref, p0_ref, p1_ref, wa_ref, ba_ref, wb_ref, bb_ref, o_ref):
        m = h_ref[...] + p0_ref[0] + p1_ref[0]
        hid = jnp.dot(m, wa_ref[...], preferred_element_type=jnp.float32) + ba_ref[...]
        hid = jnp.maximum(hid, 0.0)
        o = jnp.dot(hid, wb_ref[...], preferred_element_type=jnp.float32) + bb_ref[...]
        if relu_out:
            o = jnp.maximum(o, 0.0)
        o_ref[...] = o

    bs_rows = pl.BlockSpec((R, _D), lambda i: (i, 0))
    bs_p0 = pl.BlockSpec((1, R, _D), lambda i: (0, i, 0))
    bs_p1 = pl.BlockSpec((1, R, _D), lambda i: (1, i, 0))
    bs_w = pl.BlockSpec((_D, _D), lambda i: (0, 0))
    bs_b = pl.BlockSpec((1, _D), lambda i: (0, 0))
    return pl.pallas_call(
        body,
        grid=(_N // R,),
        in_specs=[bs_rows, bs_p0, bs_p1, bs_w, bs_b, bs_w, bs_b],
        out_specs=bs_rows,
        out_shape=jax.ShapeDtypeStruct((_N, _D), jnp.float32),
    )(h, parts, parts, Wa, ba.reshape(1, _D), Wb, bb.reshape(1, _D))


def kernel(x, edge_index, W0a, b0a, W0b, b0b, W1a, b1a, W1b, b1b, W2a, b2a, W2b, b2b):
    src = edge_index[0]
    dst = edge_index[1]
    pad = _EP - _E
    pad_i = jnp.arange(pad, dtype=jnp.int32)
    srcr = jnp.concatenate([src, pad_i % _N]).reshape(_NW, _NCH, _BE)
    dstr = jnp.concatenate([dst, _N + pad_i % (_NPAD - _N)]).reshape(_NW, _NCH, _BE)
    zeros = jnp.zeros((_NPAD, _D), jnp.float32)

    h = x
    for Wa, ba, Wb, bb, relu_out in (
        (W0a, b0a, W0b, b0b, True),
        (W1a, b1a, W1b, b1b, True),
        (W2a, b2a, W2b, b2b, False),
    ):
        parts = _segment_sum_partials(h, srcr, dstr, zeros)
        h = _mlp(h, parts, Wa, ba, Wb, bb, relu_out)
    return h
